# Initial kernel scaffold; baseline (speedup 1.0000x reference)
import functools

import jax
import jax.numpy as jnp
from jax import lax
from jax.experimental import pallas as pl
from jax.experimental.pallas import tpu as pltpu
from jax.experimental.pallas import tpu_sc as plsc

NC = 2
NS = 16
NW = NC * NS
L = 16

C = 80


@functools.lru_cache(maxsize=None)
def _encode_agg(N, D, E):
    EPW = E // NW
    NCHUNK = EPW // C
    RPT = N // NS
    mesh = plsc.VectorSubcoreMesh(core_axis_name="c", subcore_axis_name="s")

    @functools.partial(
        pl.kernel,
        mesh=mesh,
        out_type=jax.ShapeDtypeStruct((NC, N, D), jnp.float32),
        scratch_types=[
            pltpu.VMEM((C,), jnp.int32),
            pltpu.VMEM((C,), jnp.int32),
            pltpu.VMEM((C, D), jnp.float32),
            pltpu.VMEM_SHARED((N, D), jnp.float32),
            pltpu.SemaphoreType.DMA,
        ],
    )
    def k(x_hbm, src_hbm, dst_hbm, agg_hbm, idx_s, idx_d, rows, agg_sh, sem):
        cid = lax.axis_index("c")
        sid = lax.axis_index("s")
        wid = sid * NC + cid
        r0 = sid * RPT
        pltpu.sync_copy(x_hbm.at[pl.ds(r0, RPT)], agg_sh.at[pl.ds(r0, RPT)])
        plsc.subcore_barrier()
        ebase = wid * EPW

        def body(c, carry):
            b = ebase + c * C
            pltpu.sync_copy(src_hbm.at[pl.ds(b, C)], idx_s)
            pltpu.sync_copy(dst_hbm.at[pl.ds(b, C)], idx_d)
            pltpu.async_copy(x_hbm.at[idx_s], rows, sem).wait()
            pltpu.sync_copy(rows, agg_sh.at[idx_d], add=True)
            return carry

        lax.fori_loop(0, NCHUNK, body, 0)
        plsc.subcore_barrier()
        pltpu.sync_copy(agg_sh.at[pl.ds(r0, RPT)], agg_hbm.at[cid, pl.ds(r0, RPT)])

    return k


@functools.lru_cache(maxsize=None)
def _encode_mlp(N, D):
    BN = 1000

    def body(x_ref, p0_ref, p1_ref, w_ref, o_ref):
        h = p0_ref[...] + p1_ref[...] - x_ref[...]
        o_ref[...] = jnp.maximum(
            jnp.dot(h, w_ref[...], preferred_element_type=jnp.float32), 0.0
        )

    return pl.pallas_call(
        body,
        grid=(N // BN,),
        in_specs=[
            pl.BlockSpec((BN, D), lambda i: (i, 0)),
            pl.BlockSpec((BN, D), lambda i: (i, 0)),
            pl.BlockSpec((BN, D), lambda i: (i, 0)),
            pl.BlockSpec((D, D), lambda i: (0, 0)),
        ],
        out_specs=pl.BlockSpec((BN, D), lambda i: (i, 0)),
        out_shape=jax.ShapeDtypeStruct((N, D), jnp.float32),
    )


@functools.lru_cache(maxsize=None)
def _decode(N, D, E):
    EPW = E // NW
    NCHUNK = EPW // C
    G = C // L
    mesh = plsc.VectorSubcoreMesh(core_axis_name="c", subcore_axis_name="s")

    @functools.partial(
        pl.kernel,
        mesh=mesh,
        out_type=jax.ShapeDtypeStruct((E,), jnp.float32),
        scratch_types=[
            pltpu.VMEM((C,), jnp.int32),
            pltpu.VMEM((C,), jnp.int32),
            pltpu.VMEM((C, D), jnp.float32),
            pltpu.VMEM((C, D), jnp.float32),
            pltpu.VMEM((C,), jnp.float32),
            pltpu.SemaphoreType.DMA,
            pltpu.SemaphoreType.DMA,
        ],
    )
    def k(out_hbm, src_hbm, dst_hbm, pred_hbm, idx_s, idx_d, srows, trows, pv, s1, s2):
        cid = lax.axis_index("c")
        sid = lax.axis_index("s")
        wid = sid * NC + cid
        ebase = wid * EPW

        def body(c, carry):
            b = ebase + c * C
            pltpu.sync_copy(src_hbm.at[pl.ds(b, C)], idx_s)
            pltpu.sync_copy(dst_hbm.at[pl.ds(b, C)], idx_d)
            cp1 = pltpu.async_copy(out_hbm.at[idx_s], srows, s1)
            cp2 = pltpu.async_copy(out_hbm.at[idx_d], trows, s2)
            cp1.wait()
            cp2.wait()
            for g in range(G):
                eids = lax.iota(jnp.int32, L) + g * L
                acc = jnp.zeros((L,), jnp.float32)
                for d in range(D):
                    dv = jnp.full((L,), d, jnp.int32)
                    sv = plsc.load_gather(srows, [eids, dv])
                    tv = plsc.load_gather(trows, [eids, dv])
                    acc = acc + sv * tv
                pv[pl.ds(g * L, L)] = acc
            pltpu.sync_copy(pv, pred_hbm.at[pl.ds(b, C)])
            return carry

        lax.fori_loop(0, NCHUNK, body, 0)

    return k


def kernel(x, edge_index, W):
    N, D = x.shape
    E = edge_index.shape[1]
    assert E % (NW * C) == 0 and N % NS == 0
    src = edge_index[0]
    dst = edge_index[1]
    agg2 = _encode_agg(N, D, E)(x, src, dst)
    out = _encode_mlp(N, D)(x, agg2[0], agg2[1], W)
    return _decode(N, D, E)(out, src, dst)


# speedup vs baseline: 3.1666x; 3.1666x over previous
import functools

import jax
import jax.numpy as jnp
from jax import lax
from jax.experimental import pallas as pl
from jax.experimental.pallas import tpu as pltpu
from jax.experimental.pallas import tpu_sc as plsc

NC = 2
NS = 16
NW = NC * NS
L = 16

C = 80


@functools.lru_cache(maxsize=None)
def _encode_agg(N, D, E):
    EPW = E // NW
    NCHUNK = EPW // C
    RPT = (N // NS) // 8 * 8
    TAIL = N - RPT * NS
    mesh = plsc.VectorSubcoreMesh(core_axis_name="c", subcore_axis_name="s")

    @functools.partial(
        pl.kernel,
        mesh=mesh,
        compiler_params=pltpu.CompilerParams(needs_layout_passes=False),
        out_type=jax.ShapeDtypeStruct((NC, N, D), jnp.float32),
        scratch_types=[
            pltpu.VMEM((C,), jnp.int32),
            pltpu.VMEM((C,), jnp.int32),
            pltpu.VMEM((C, D), jnp.float32),
            pltpu.VMEM_SHARED((N, D), jnp.float32),
            pltpu.SemaphoreType.DMA,
        ],
    )
    def k(x_hbm, src_hbm, dst_hbm, agg_hbm, idx_s, idx_d, rows, agg_sh, sem):
        cid = lax.axis_index("c")
        sid = lax.axis_index("s")
        wid = sid * NC + cid
        r0 = sid * RPT
        pltpu.sync_copy(x_hbm.at[pl.ds(r0, RPT)], agg_sh.at[pl.ds(r0, RPT)])
        @pl.when(sid == 0)
        def _():
            pltpu.sync_copy(
                x_hbm.at[pl.ds(RPT * NS, TAIL)], agg_sh.at[pl.ds(RPT * NS, TAIL)]
            )
        plsc.subcore_barrier()
        ebase = wid * EPW

        def body(c, carry):
            b = ebase + c * C
            pltpu.sync_copy(src_hbm.at[pl.ds(b, C)], idx_s)
            pltpu.sync_copy(dst_hbm.at[pl.ds(b, C)], idx_d)
            pltpu.async_copy(x_hbm.at[idx_s], rows, sem).wait()
            pltpu.sync_copy(rows, agg_sh.at[idx_d], add=True)
            return carry

        lax.fori_loop(0, NCHUNK, body, 0)
        plsc.subcore_barrier()
        pltpu.sync_copy(agg_sh.at[pl.ds(r0, RPT)], agg_hbm.at[cid, pl.ds(r0, RPT)])
        @pl.when(sid == 0)
        def _():
            pltpu.sync_copy(
                agg_sh.at[pl.ds(RPT * NS, TAIL)],
                agg_hbm.at[cid, pl.ds(RPT * NS, TAIL)],
            )

    return k


@functools.lru_cache(maxsize=None)
def _encode_mlp(N, D):
    BN = 1000

    def body(x_ref, p0_ref, p1_ref, w_ref, o_ref):
        h = p0_ref[...] + p1_ref[...] - x_ref[...]
        o_ref[...] = jnp.maximum(
            jnp.dot(h, w_ref[...], preferred_element_type=jnp.float32), 0.0
        )

    return pl.pallas_call(
        body,
        grid=(N // BN,),
        in_specs=[
            pl.BlockSpec((BN, D), lambda i: (i, 0)),
            pl.BlockSpec((BN, D), lambda i: (i, 0)),
            pl.BlockSpec((BN, D), lambda i: (i, 0)),
            pl.BlockSpec((D, D), lambda i: (0, 0)),
        ],
        out_specs=pl.BlockSpec((BN, D), lambda i: (i, 0)),
        out_shape=jax.ShapeDtypeStruct((N, D), jnp.float32),
    )


@functools.lru_cache(maxsize=None)
def _decode(N, D, E):
    EPW = E // NW
    NCHUNK = EPW // C
    G = C // L
    mesh = plsc.VectorSubcoreMesh(core_axis_name="c", subcore_axis_name="s")

    @functools.partial(
        pl.kernel,
        mesh=mesh,
        compiler_params=pltpu.CompilerParams(needs_layout_passes=False),
        out_type=jax.ShapeDtypeStruct((E,), jnp.float32),
        scratch_types=[
            pltpu.VMEM((C,), jnp.int32),
            pltpu.VMEM((C,), jnp.int32),
            pltpu.VMEM((C, D), jnp.float32),
            pltpu.VMEM((C, D), jnp.float32),
            pltpu.VMEM((C,), jnp.float32),
            pltpu.SemaphoreType.DMA,
            pltpu.SemaphoreType.DMA,
        ],
    )
    def k(out_hbm, src_hbm, dst_hbm, pred_hbm, idx_s, idx_d, srows, trows, pv, s1, s2):
        cid = lax.axis_index("c")
        sid = lax.axis_index("s")
        wid = sid * NC + cid
        ebase = wid * EPW

        def body(c, carry):
            b = ebase + c * C
            pltpu.sync_copy(src_hbm.at[pl.ds(b, C)], idx_s)
            pltpu.sync_copy(dst_hbm.at[pl.ds(b, C)], idx_d)
            cp1 = pltpu.async_copy(out_hbm.at[idx_s], srows, s1)
            cp2 = pltpu.async_copy(out_hbm.at[idx_d], trows, s2)
            cp1.wait()
            cp2.wait()
            lane = lax.iota(jnp.int32, L)
            for g in range(G):
                res = jnp.zeros((L,), jnp.float32)
                for j in range(L):
                    e = g * L + j
                    acc = srows[e, pl.ds(0, L)] * trows[e, pl.ds(0, L)]
                    for k in range(1, D // L):
                        acc = acc + srows[e, pl.ds(k * L, L)] * trows[e, pl.ds(k * L, L)]
                    res = jnp.where(lane == j, jnp.sum(acc), res)
                pv[pl.ds(g * L, L)] = res
            pltpu.sync_copy(pv, pred_hbm.at[pl.ds(b, C)])
            return carry

        lax.fori_loop(0, NCHUNK, body, 0)

    return k


def kernel(x, edge_index, W):
    N, D = x.shape
    E = edge_index.shape[1]
    assert E % (NW * C) == 0 and N % NS == 0
    src = edge_index[0]
    dst = edge_index[1]
    agg2 = _encode_agg(N, D, E)(x, src, dst)
    out = _encode_mlp(N, D)(x, agg2[0], agg2[1], W)
    return _decode(N, D, E)(out, src, dst)
